# baseline (speedup 1.0000x reference)
import functools

import jax
import jax.numpy as jnp
from jax import lax
from jax.experimental import pallas as pl
from jax.experimental.pallas import tpu as pltpu
from jax.experimental.pallas import tpu_sc as plsc

_N = 16384
_C = 1000
_NC = 2
_NS = 16
_NW = _NC * _NS
_RPW = _N // _NW
_L = 16
_GCH = 128
_NG = _RPW // _GCH

_mesh = plsc.VectorSubcoreMesh(core_axis_name="c", subcore_axis_name="s")


@functools.partial(
    pl.kernel,
    mesh=_mesh,
    out_type=jax.ShapeDtypeStruct((_NC, _L), jnp.float32),
    scratch_types=[
        pltpu.VMEM((_RPW,), jnp.int32),
        pltpu.VMEM((_RPW,), jnp.float32),
        pltpu.VMEM((_NG, _GCH), jnp.int32),
        pltpu.VMEM((_NG, _GCH), jnp.float32),
        pltpu.VMEM((_L,), jnp.float32),
        pltpu.VMEM((_NS * _L,), jnp.float32),
        pltpu.VMEM_SHARED((_NS * _L,), jnp.float32),
        pltpu.SemaphoreType.DMA,
    ],
)
def _pg_loss_sc(pred_hbm, tgt_hbm, rew_hbm, out_hbm,
                tgt_v, rew_v, idx_v, val_v, st_v, allp_v, shared, sem):
    cid = lax.axis_index("c")
    sid = lax.axis_index("s")
    wid = sid * _NC + cid
    base = wid * _RPW

    pltpu.sync_copy(tgt_hbm.at[pl.ds(base, _RPW)], tgt_v)
    pltpu.sync_copy(rew_hbm.at[pl.ds(base, _RPW)], rew_v)

    lane = lax.iota(jnp.int32, _L)
    for ch in range(_RPW // _L):
        t16 = tgt_v[pl.ds(ch * _L, _L)]
        rows = base + ch * _L + lane
        g, off = ch // (_GCH // _L), (ch % (_GCH // _L)) * _L
        idx_v[g, pl.ds(off, _L)] = rows * _C + t16

    copies = [
        pltpu.async_copy(pred_hbm.at[idx_v.at[g]], val_v.at[g], sem)
        for g in range(_NG)
    ]
    for cp in copies:
        cp.wait()

    acc = jnp.zeros((_L,), jnp.float32)
    for ch in range(_RPW // _L):
        g, off = ch // (_GCH // _L), (ch % (_GCH // _L)) * _L
        acc = acc + val_v[g, pl.ds(off, _L)] * rew_v[pl.ds(ch * _L, _L)]

    st_v[...] = acc
    pltpu.sync_copy(st_v, shared.at[pl.ds(sid * _L, _L)])
    plsc.subcore_barrier()

    @pl.when(sid == 0)
    def _():
        pltpu.sync_copy(shared, allp_v)
        tot = jnp.zeros((_L,), jnp.float32)
        for s in range(_NS):
            tot = tot + allp_v[pl.ds(s * _L, _L)]
        st_v[...] = -tot
        pltpu.sync_copy(st_v, out_hbm.at[cid])


def kernel(pred, target, reward):
    pred_flat = pred.reshape(-1)
    tgt = target.astype(jnp.int32)
    rew = reward.astype(jnp.float32)
    partials = _pg_loss_sc(pred_flat, tgt, rew)
    return jnp.sum(partials)


# speedup vs baseline: 1.4329x; 1.4329x over previous
import functools

import jax
import jax.numpy as jnp
from jax import lax
from jax.experimental import pallas as pl
from jax.experimental.pallas import tpu as pltpu
from jax.experimental.pallas import tpu_sc as plsc

_N = 16384
_C = 1000
_NC = 2
_NS = 16
_NW = _NC * _NS
_RPW = _N // _NW
_L = 16
_BR = 32
_NCH = _RPW // _BR

_mesh = plsc.VectorSubcoreMesh(core_axis_name="c", subcore_axis_name="s")


@functools.partial(
    pl.kernel,
    mesh=_mesh,
    out_type=jax.ShapeDtypeStruct((_NC, _L), jnp.float32),
    scratch_types=[
        pltpu.VMEM((_RPW,), jnp.int32),
        pltpu.VMEM((_RPW,), jnp.float32),
        pltpu.VMEM((2, _BR, _C), jnp.float32),
        pltpu.VMEM((_L,), jnp.float32),
        pltpu.VMEM((_NS * _L,), jnp.float32),
        pltpu.VMEM_SHARED((_NS * _L,), jnp.float32),
        pltpu.SemaphoreType.DMA,
        pltpu.SemaphoreType.DMA,
    ],
)
def _pg_loss_sc(pred_hbm, tgt_hbm, rew_hbm, out_hbm,
                tgt_v, rew_v, buf_v, st_v, allp_v, shared,
                sem0, sem1):
    cid = lax.axis_index("c")
    sid = lax.axis_index("s")
    wid = sid * _NC + cid
    base = wid * _RPW

    pltpu.sync_copy(tgt_hbm.at[pl.ds(base, _RPW)], tgt_v)
    pltpu.sync_copy(rew_hbm.at[pl.ds(base, _RPW)], rew_v)

    sems = (sem0, sem1)

    def start(ch, slot):
        pltpu.async_copy(
            pred_hbm.at[pl.ds(base + ch * _BR, _BR), :],
            buf_v.at[slot], sems[slot])

    def drain(slot):
        pltpu.make_async_copy(
            pred_hbm.at[pl.ds(base, _BR), :],
            buf_v.at[slot], sems[slot]).wait()

    lane = lax.iota(jnp.int32, _L)

    def fold(ch, slot, acc):
        for q in range(_BR // _L):
            t16 = tgt_v[pl.ds(ch * _BR + q * _L, _L)]
            w16 = rew_v[pl.ds(ch * _BR + q * _L, _L)]
            for u in range(_L):
                j = q * _L + u
                t = t16[u]
                c16 = pl.multiple_of((t >> 4) << 4, _L)
                p = t & 15
                v16 = buf_v[slot, j, pl.ds(c16, _L)]
                acc = acc + jnp.where(lane == p, v16, 0.0) * w16[u]
        return acc

    start(0, 0)
    start(1, 1)

    def body(i, acc):
        ch0 = i * 2
        drain(0)
        acc = fold(ch0, 0, acc)

        @pl.when(ch0 + 2 < _NCH)
        def _():
            start(ch0 + 2, 0)

        drain(1)
        acc = fold(ch0 + 1, 1, acc)

        @pl.when(ch0 + 3 < _NCH)
        def _():
            start(ch0 + 3, 1)

        return acc

    acc = lax.fori_loop(0, _NCH // 2, body, jnp.zeros((_L,), jnp.float32),
                        unroll=False)

    st_v[...] = acc
    pltpu.sync_copy(st_v, shared.at[pl.ds(sid * _L, _L)])
    plsc.subcore_barrier()

    @pl.when(sid == 0)
    def _():
        pltpu.sync_copy(shared, allp_v)
        tot = jnp.zeros((_L,), jnp.float32)
        for s in range(_NS):
            tot = tot + allp_v[pl.ds(s * _L, _L)]
        st_v[...] = -tot
        pltpu.sync_copy(st_v, out_hbm.at[cid])


def kernel(pred, target, reward):
    tgt = target.astype(jnp.int32)
    rew = reward.astype(jnp.float32)
    partials = _pg_loss_sc(pred, tgt, rew)
    return jnp.sum(partials)


# speedup vs baseline: 1.4402x; 1.0051x over previous
import functools

import jax
import jax.numpy as jnp
from jax import lax
from jax.experimental import pallas as pl
from jax.experimental.pallas import tpu as pltpu
from jax.experimental.pallas import tpu_sc as plsc

_N = 16384
_C = 1000
_NC = 2
_NS = 16
_NW = _NC * _NS
_RPW = _N // _NW
_L = 16
_BR = 32
_NCH = _RPW // _BR

_mesh = plsc.VectorSubcoreMesh(core_axis_name="c", subcore_axis_name="s")


@functools.partial(
    pl.kernel,
    mesh=_mesh,
    out_type=jax.ShapeDtypeStruct((_NW, _L), jnp.float32),
    scratch_types=[
        pltpu.VMEM((_RPW,), jnp.int32),
        pltpu.VMEM((_RPW,), jnp.float32),
        pltpu.VMEM((2, _BR, _C), jnp.float32),
        pltpu.VMEM((_L,), jnp.float32),
        pltpu.SemaphoreType.DMA,
        pltpu.SemaphoreType.DMA,
    ],
)
def _pg_loss_sc(pred_hbm, tgt_hbm, rew_hbm, out_hbm,
                tgt_v, rew_v, buf_v, st_v,
                sem0, sem1):
    cid = lax.axis_index("c")
    sid = lax.axis_index("s")
    wid = sid * _NC + cid
    base = wid * _RPW

    pltpu.sync_copy(tgt_hbm.at[pl.ds(base, _RPW)], tgt_v)
    pltpu.sync_copy(rew_hbm.at[pl.ds(base, _RPW)], rew_v)

    sems = (sem0, sem1)

    def start(ch, slot):
        pltpu.async_copy(
            pred_hbm.at[pl.ds(base + ch * _BR, _BR), :],
            buf_v.at[slot], sems[slot])

    def drain(slot):
        pltpu.make_async_copy(
            pred_hbm.at[pl.ds(base, _BR), :],
            buf_v.at[slot], sems[slot]).wait()

    lane = lax.iota(jnp.int32, _L)

    def fold(ch, slot, acc):
        for q in range(_BR // _L):
            t16 = tgt_v[pl.ds(ch * _BR + q * _L, _L)]
            w16 = rew_v[pl.ds(ch * _BR + q * _L, _L)]
            for u in range(_L):
                j = q * _L + u
                t = t16[u]
                c16 = pl.multiple_of((t >> 4) << 4, _L)
                p = t & 15
                v16 = buf_v[slot, j, pl.ds(c16, _L)]
                acc = acc + jnp.where(lane == p, v16, 0.0) * w16[u]
        return acc

    start(0, 0)
    start(1, 1)

    def body(i, acc):
        ch0 = i * 2
        drain(0)
        acc = fold(ch0, 0, acc)

        @pl.when(ch0 + 2 < _NCH)
        def _():
            start(ch0 + 2, 0)

        drain(1)
        acc = fold(ch0 + 1, 1, acc)

        @pl.when(ch0 + 3 < _NCH)
        def _():
            start(ch0 + 3, 1)

        return acc

    acc = lax.fori_loop(0, _NCH // 2, body, jnp.zeros((_L,), jnp.float32),
                        unroll=False)

    st_v[...] = -acc
    pltpu.sync_copy(st_v, out_hbm.at[wid])


def kernel(pred, target, reward):
    tgt = target.astype(jnp.int32)
    rew = reward.astype(jnp.float32)
    partials = _pg_loss_sc(pred, tgt, rew)
    return jnp.sum(partials)
